# baseline (speedup 1.0000x reference)
import math
import jax
import jax.numpy as jnp
from jax import lax
from jax.experimental import pallas as pl
from jax.experimental.pallas import tpu as pltpu
from jax.experimental.pallas import tpu_sc as plsc

N = 10000
E = 160000
B = 2
D = 128
R = 16
TDIM = 64
OUT = 128
HALF_LIFE = 200.0
EPS = 1e-06

NB = 32
SUB = NB // 16
NBLK = (N + NB - 1) // NB
NPAD = NBLK * NB
NC, NS = 1, 16
NW = NC * NS
WBLKS = (NBLK + NW - 1) // NW
G = 128
CH = 640
EP = E + CH
NEG = -3.0e38


def _edge_prep_body(tc_ref, dc_ref, yc_ref, qt_ref, wt_ref, bt_ref,
                    q_ref, wr_ref, br_ref, tpm_ref, rel_ref):
    pid = pl.program_id(0)
    t = tc_ref[...]
    iot = lax.broadcasted_iota(jnp.int32, (1, TDIM // 2), 1).astype(jnp.float32)
    invf = jnp.exp(-iot * (math.log(10000.0) / (TDIM // 2 - 1)))
    ang = t * invf
    te = jnp.concatenate([jnp.sin(ang), jnp.cos(ang)], axis=1)
    tp = jnp.dot(te, wt_ref[...],
                 preferred_element_type=jnp.float32) + bt_ref[...]
    qtv = qt_ref[...].reshape(1, 8)
    cols = [tp, dc_ref[...], yc_ref[...]]
    for b in range(B):
        dt = jnp.full((CH, 1), qtv[0, b], jnp.float32) - t
        cols.append(jnp.exp(dt * (-math.log(2.0) / HALF_LIFE)) * (dt >= 0.0))
    cols.append(jnp.zeros((CH, B * D - D - 4), jnp.float32))
    tpm_ref[...] = jnp.concatenate(cols, axis=1)

    @pl.when(pid == 0)
    def _():
        qW = jnp.dot(q_ref[...], wr_ref[...],
                     preferred_element_type=jnp.float32)
        rel_rows = []
        for r in range(R):
            sl = qW[:, r * D:(r + 1) * D] + br_ref[r:r + 1, :]
            rel_rows.append(jnp.concatenate([sl[0:1], sl[1:2]], axis=1))
        rel_ref[...] = jnp.concatenate(rel_rows, axis=0)


def _edge_prep(tcol, dcol, ycol, qt3, W_time, b_time2, query, W_rel, b_rel2):
    nchunks = EP // CH
    col = pl.BlockSpec((CH, 1), lambda i: (i, 0))
    return pl.pallas_call(
        _edge_prep_body,
        grid=(nchunks,),
        in_specs=[
            col, col, col,
            pl.BlockSpec((1, 1, 8), lambda i: (0, 0, 0)),
            pl.BlockSpec((TDIM, D), lambda i: (0, 0)),
            pl.BlockSpec((1, D), lambda i: (0, 0)),
            pl.BlockSpec((B, D), lambda i: (0, 0)),
            pl.BlockSpec((D, R * D), lambda i: (0, 0)),
            pl.BlockSpec((R, D), lambda i: (0, 0)),
        ],
        out_specs=[
            pl.BlockSpec((CH, B * D), lambda i: (i, 0)),
            pl.BlockSpec((R, B * D), lambda i: (0, 0)),
        ],
        out_shape=[
            jax.ShapeDtypeStruct((EP, B * D), jnp.float32),
            jax.ShapeDtypeStruct((R, B * D), jnp.float32),
        ],
    )(tcol, dcol, ycol, qt3, W_time, b_time2, query, W_rel, b_rel2)


def _sc_body(xr, rel, srcp, tpm, blkb,
             zer, ninf, pinf,
             osum, osq, omax, omin,
             accs, accq, accmx, accmn,
             xbuf, tpbuf, metab, srcb, bv, sem):
    wid = lax.axis_index("s") * NC + lax.axis_index("c")
    pltpu.sync_copy(rel, metab.at[pl.ds(16, R), :])
    H = D // 2

    def do_block(nb):
        pltpu.sync_copy(blkb.at[nb], bv)
        b2 = bv[...]
        e_begin = b2[0]
        e_end = b2[1]
        pltpu.sync_copy(zer, accs)
        pltpu.sync_copy(zer, accq)
        pltpu.sync_copy(ninf, accmx)
        pltpu.sync_copy(pinf, accmn)
        c0 = e_begin // G
        c1 = (e_end + (G - 1)) // G

        def do_chunk(ch, _):
            g0 = ch * G
            pltpu.sync_copy(srcp.at[pl.ds(g0, G)], srcb)
            pltpu.sync_copy(tpm.at[pl.ds(g0, G)], tpbuf)
            pltpu.async_copy(xr.at[srcb], xbuf, sem).wait()
            lo = jnp.maximum(e_begin - g0, 0)
            hi = jnp.minimum(e_end - g0, G)

            def do_edge(e, _):
                mv = tpbuf[e, pl.ds(D, 16)]
                dl = mv[0].astype(jnp.int32)
                ty = mv[1].astype(jnp.int32) + 16
                w0 = mv[2]
                w1 = mv[3]
                for kre in range(H // 16):
                    cr = kre * 16
                    ci = cr + H
                    tpr = tpbuf[e, pl.ds(cr, 16)]
                    tpi = tpbuf[e, pl.ds(ci, 16)]
                    for b in range(B):
                        o = b * D
                        xre = xbuf[e, pl.ds(o + cr, 16)]
                        xim = xbuf[e, pl.ds(o + ci, 16)]
                        ere = metab[ty, pl.ds(o + cr, 16)]
                        eim = metab[ty, pl.ds(o + ci, 16)]
                        w = w0 if b == 0 else w1
                        mre = (xre * ere - xim * eim + tpr) * w
                        mim = (xre * eim + xim * ere + tpi) * w
                        for c, v in ((o + cr, mre), (o + ci, mim)):
                            accs[dl, pl.ds(c, 16)] = accs[dl, pl.ds(c, 16)] + v
                            accq[dl, pl.ds(c, 16)] = accq[dl, pl.ds(c, 16)] + v * v
                            accmx[dl, pl.ds(c, 16)] = jnp.maximum(
                                accmx[dl, pl.ds(c, 16)], v)
                            accmn[dl, pl.ds(c, 16)] = jnp.minimum(
                                accmn[dl, pl.ds(c, 16)], v)
                return 0

            lax.fori_loop(lo, hi, do_edge, 0)
            return 0

        lax.fori_loop(c0, c1, do_chunk, 0)
        pltpu.sync_copy(accs, osum.at[pl.ds(nb * NB, NB)])
        pltpu.sync_copy(accq, osq.at[pl.ds(nb * NB, NB)])
        pltpu.sync_copy(accmx, omax.at[pl.ds(nb * NB, NB)])
        pltpu.sync_copy(accmn, omin.at[pl.ds(nb * NB, NB)])

    def blk_loop(i, _):
        nb = wid + NW * i

        @pl.when(nb < NBLK)
        def _():
            do_block(nb)
        return 0

    lax.fori_loop(0, WBLKS, blk_loop, 0)


def _sc_agg(xr, rel, srcp, tpm, blkb):
    zer = jnp.zeros((NB, B * D), jnp.float32)
    ninf = jnp.full((NB, B * D), NEG, jnp.float32)
    pinf = jnp.full((NB, B * D), -NEG, jnp.float32)
    mesh = plsc.VectorSubcoreMesh(core_axis_name="c", subcore_axis_name="s",
                                  num_cores=1)
    f = pl.kernel(
        _sc_body,
        out_type=[
            jax.ShapeDtypeStruct((NPAD, B * D), jnp.float32),
            jax.ShapeDtypeStruct((NPAD, B * D), jnp.float32),
            jax.ShapeDtypeStruct((NPAD, B * D), jnp.float32),
            jax.ShapeDtypeStruct((NPAD, B * D), jnp.float32),
        ],
        mesh=mesh,
        scratch_types=[
            pltpu.VMEM((NB, B * D), jnp.float32),
            pltpu.VMEM((NB, B * D), jnp.float32),
            pltpu.VMEM((NB, B * D), jnp.float32),
            pltpu.VMEM((NB, B * D), jnp.float32),
            pltpu.VMEM((G, B * D), jnp.float32),
            pltpu.VMEM((G, B * D), jnp.float32),
            pltpu.VMEM((64, B * D), jnp.float32),
            pltpu.VMEM((G,), jnp.int32),
            pltpu.VMEM((16,), jnp.int32),
            pltpu.SemaphoreType.DMA,
        ],
    )
    return f(xr, rel, srcp, tpm, blkb, zer, ninf, pinf)


def _combine_body(osum_ref, osq_ref, omax_ref, omin_ref, scl_ref,
                  x_ref, bnd_ref, W_ref, bl_ref, out0_ref, out1_ref):
    bnd = bnd_ref[...]
    c = scl_ref[:, 0:1]
    s = osum_ref[...] + bnd
    q = osq_ref[...] + bnd * bnd
    mx = jnp.maximum(omax_ref[...], bnd)
    mn = jnp.minimum(omin_ref[...], bnd)
    mean = s / c
    std = jnp.sqrt(jnp.maximum(q / c - mean * mean, EPS))
    sc1 = scl_ref[:, 1:2]
    sc2 = scl_ref[:, 2:3]
    for b, oref in ((0, out0_ref), (1, out1_ref)):
        sl = slice(b * D, (b + 1) * D)
        parts = []
        for st in (mean, mx, mn, std):
            stb = st[:, sl]
            parts.extend([stb, stb * sc1, stb * sc2])
        z = jnp.concatenate(parts + [x_ref[:, sl]], axis=1)
        ob = jnp.dot(z, W_ref[...], preferred_element_type=jnp.float32)
        oref[...] = jnp.maximum(ob + bl_ref[...], 0.0)


def _combine(osum, osq, omax, omin, scl, xpad, bpad, W_perm, b_lin2):
    blk = pl.BlockSpec((NB, B * D), lambda i: (i, 0))
    return pl.pallas_call(
        _combine_body,
        grid=(NBLK,),
        in_specs=[
            blk, blk, blk, blk,
            pl.BlockSpec((NB, 8), lambda i: (i, 0)),
            blk, blk,
            pl.BlockSpec(((12 + 1) * D, OUT), lambda i: (0, 0)),
            pl.BlockSpec((1, OUT), lambda i: (0, 0)),
        ],
        out_specs=[
            pl.BlockSpec((NB, OUT), lambda i: (i, 0)),
            pl.BlockSpec((NB, OUT), lambda i: (i, 0)),
        ],
        out_shape=[
            jax.ShapeDtypeStruct((NPAD, OUT), jnp.float32),
            jax.ShapeDtypeStruct((NPAD, OUT), jnp.float32),
        ],
    )(osum, osq, omax, omin, scl, xpad, bpad, W_perm, b_lin2)


def kernel(x, boundary, query, edge_weight, edge_index, edge_type, edge_time,
           query_time, W_rel, b_rel, W_time, b_time, W_lin, b_lin):
    src = edge_index[0].astype(jnp.int32)
    dst = edge_index[1].astype(jnp.int32)
    et = edge_type.astype(jnp.int32)
    tm = edge_time.astype(jnp.int32)

    perm = jnp.argsort(dst)
    srcp = jnp.pad(src[perm], (0, EP - E))
    dstp = dst[perm]
    typep = jnp.pad(et[perm], (0, EP - E))
    tmp_ = jnp.pad(tm[perm], (0, EP - E))
    dstloc = jnp.pad(dstp % NB, (0, EP - E))
    blkb1 = jnp.searchsorted(dstp, jnp.arange(0, NPAD + NB, NB)).astype(jnp.int32)
    blkb = jnp.stack([blkb1[:NBLK], blkb1[1:NBLK + 1]], axis=1)
    blkb = jnp.pad(blkb, ((0, 0), (0, 14)))

    jp = jnp.arange(12 * D)
    k_, s_, d_ = jp // (3 * D), (jp // D) % 3, jp % D
    W_perm = jnp.concatenate([W_lin[d_ * 12 + k_ * 3 + s_], W_lin[12 * D:]], axis=0)

    tcol = tmp_.astype(jnp.float32).reshape(EP, 1)
    dcol = dstloc.astype(jnp.float32).reshape(EP, 1)
    ycol = typep.astype(jnp.float32).reshape(EP, 1)
    qt3 = jnp.pad(query_time.astype(jnp.float32), (0, 8 - B)).reshape(1, 1, 8)
    b_time2 = b_time.reshape(1, D)
    b_rel2 = b_rel.reshape(R, D)

    tpm, rel = _edge_prep(tcol, dcol, ycol, qt3, W_time, b_time2,
                          query, W_rel, b_rel2)

    xr = x.reshape(N, B * D)
    osum, osq, omax, omin = _sc_agg(xr, rel, srcp, tpm, blkb)

    node_b = jnp.searchsorted(dstp, jnp.arange(0, N + 1)).astype(jnp.int32)
    cnt = (node_b[1:] - node_b[:-1]).astype(jnp.float32) + 1.0
    logd = jnp.log(cnt)
    scale = logd / jnp.mean(logd)
    inv = 1.0 / jnp.maximum(scale, 1e-2)
    scl = jnp.zeros((NPAD, 8), jnp.float32)
    scl = scl.at[:, 0].set(1.0)
    scl = scl.at[:N, 0].set(cnt)
    scl = scl.at[:N, 1].set(scale)
    scl = scl.at[:N, 2].set(inv)

    xpad = jnp.pad(xr, ((0, NPAD - N), (0, 0)))
    bpad = jnp.pad(boundary.reshape(N, B * D), ((0, NPAD - N), (0, 0)))
    out0, out1 = _combine(osum, osq, omax, omin, scl, xpad, bpad,
                          W_perm, b_lin.reshape(1, OUT))
    return jnp.stack([out0[:N], out1[:N]], axis=1)


# speedup vs baseline: 1.3624x; 1.3624x over previous
import math
import jax
import jax.numpy as jnp
from jax import lax
from jax.experimental import pallas as pl
from jax.experimental.pallas import tpu as pltpu
from jax.experimental.pallas import tpu_sc as plsc

N = 10000
E = 160000
B = 2
D = 128
R = 16
TDIM = 64
OUT = 128
HALF_LIFE = 200.0
EPS = 1e-06

NB = 32
SUB = NB // 16
NBLK = (N + NB - 1) // NB
NPAD = NBLK * NB
NC, NS = 2, 16
NW = NC * NS
WBLKS = (NBLK + NW - 1) // NW
G = 128
CH = 640
EP = E + CH
NEG = -3.0e38


def _edge_prep_body(tc_ref, dc_ref, yc_ref, qt_ref, wt_ref, bt_ref,
                    q_ref, wr_ref, br_ref, tpm_ref, rel_ref):
    pid = pl.program_id(0)
    t = tc_ref[...]
    iot = lax.broadcasted_iota(jnp.int32, (1, TDIM // 2), 1).astype(jnp.float32)
    invf = jnp.exp(-iot * (math.log(10000.0) / (TDIM // 2 - 1)))
    ang = t * invf
    te = jnp.concatenate([jnp.sin(ang), jnp.cos(ang)], axis=1)
    tp = jnp.dot(te, wt_ref[...],
                 preferred_element_type=jnp.float32) + bt_ref[...]
    qtv = qt_ref[...].reshape(1, 8)
    cols = [tp, dc_ref[...], yc_ref[...]]
    for b in range(B):
        dt = jnp.full((CH, 1), qtv[0, b], jnp.float32) - t
        cols.append(jnp.exp(dt * (-math.log(2.0) / HALF_LIFE)) * (dt >= 0.0))
    cols.append(jnp.zeros((CH, B * D - D - 4), jnp.float32))
    tpm_ref[...] = jnp.concatenate(cols, axis=1)

    @pl.when(pid == 0)
    def _():
        qW = jnp.dot(q_ref[...], wr_ref[...],
                     preferred_element_type=jnp.float32)
        rel_rows = []
        for r in range(R):
            sl = qW[:, r * D:(r + 1) * D] + br_ref[r:r + 1, :]
            rel_rows.append(jnp.concatenate([sl[0:1], sl[1:2]], axis=1))
        rel_ref[...] = jnp.concatenate(rel_rows, axis=0)


def _edge_prep(tcol, dcol, ycol, qt3, W_time, b_time2, query, W_rel, b_rel2):
    nchunks = EP // CH
    col = pl.BlockSpec((CH, 1), lambda i: (i, 0))
    return pl.pallas_call(
        _edge_prep_body,
        grid=(nchunks,),
        in_specs=[
            col, col, col,
            pl.BlockSpec((1, 1, 8), lambda i: (0, 0, 0)),
            pl.BlockSpec((TDIM, D), lambda i: (0, 0)),
            pl.BlockSpec((1, D), lambda i: (0, 0)),
            pl.BlockSpec((B, D), lambda i: (0, 0)),
            pl.BlockSpec((D, R * D), lambda i: (0, 0)),
            pl.BlockSpec((R, D), lambda i: (0, 0)),
        ],
        out_specs=[
            pl.BlockSpec((CH, B * D), lambda i: (i, 0)),
            pl.BlockSpec((R, B * D), lambda i: (0, 0)),
        ],
        out_shape=[
            jax.ShapeDtypeStruct((EP, B * D), jnp.float32),
            jax.ShapeDtypeStruct((R, B * D), jnp.float32),
        ],
    )(tcol, dcol, ycol, qt3, W_time, b_time2, query, W_rel, b_rel2)


def _sc_body(xr, rel, srcp, tpm, blkb,
             zer, ninf, pinf,
             osum, osq, omax, omin,
             accs, accq, accmx, accmn,
             xbuf, tpbuf, metab, srcb, bv, sem):
    wid = lax.axis_index("s") * NC + lax.axis_index("c")
    pltpu.sync_copy(rel, metab.at[pl.ds(16, R), :])
    H = D // 2

    def do_block(nb):
        pltpu.sync_copy(blkb.at[nb], bv)
        b2 = bv[...]
        e_begin = b2[0]
        e_end = b2[1]
        pltpu.sync_copy(zer, accs)
        pltpu.sync_copy(zer, accq)
        pltpu.sync_copy(ninf, accmx)
        pltpu.sync_copy(pinf, accmn)
        c0 = e_begin // G
        c1 = (e_end + (G - 1)) // G

        def do_chunk(ch, _):
            g0 = ch * G
            pltpu.sync_copy(srcp.at[pl.ds(g0, G)], srcb)
            pltpu.sync_copy(tpm.at[pl.ds(g0, G)], tpbuf)
            pltpu.async_copy(xr.at[srcb], xbuf, sem).wait()
            lo = jnp.maximum(e_begin - g0, 0)
            hi = jnp.minimum(e_end - g0, G)

            def do_edge(e, _):
                mv = tpbuf[e, pl.ds(D, 16)]
                dl = mv[0].astype(jnp.int32)
                ty = mv[1].astype(jnp.int32) + 16
                w0 = mv[2]
                w1 = mv[3]
                for kre in range(H // 16):
                    cr = kre * 16
                    ci = cr + H
                    tpr = tpbuf[e, pl.ds(cr, 16)]
                    tpi = tpbuf[e, pl.ds(ci, 16)]
                    for b in range(B):
                        o = b * D
                        xre = xbuf[e, pl.ds(o + cr, 16)]
                        xim = xbuf[e, pl.ds(o + ci, 16)]
                        ere = metab[ty, pl.ds(o + cr, 16)]
                        eim = metab[ty, pl.ds(o + ci, 16)]
                        w = w0 if b == 0 else w1
                        mre = (xre * ere - xim * eim + tpr) * w
                        mim = (xre * eim + xim * ere + tpi) * w
                        for c, v in ((o + cr, mre), (o + ci, mim)):
                            accs[dl, pl.ds(c, 16)] = accs[dl, pl.ds(c, 16)] + v
                            accq[dl, pl.ds(c, 16)] = accq[dl, pl.ds(c, 16)] + v * v
                            accmx[dl, pl.ds(c, 16)] = jnp.maximum(
                                accmx[dl, pl.ds(c, 16)], v)
                            accmn[dl, pl.ds(c, 16)] = jnp.minimum(
                                accmn[dl, pl.ds(c, 16)], v)
                return 0

            lax.fori_loop(lo, hi, do_edge, 0)
            return 0

        lax.fori_loop(c0, c1, do_chunk, 0)
        pltpu.sync_copy(accs, osum.at[pl.ds(nb * NB, NB)])
        pltpu.sync_copy(accq, osq.at[pl.ds(nb * NB, NB)])
        pltpu.sync_copy(accmx, omax.at[pl.ds(nb * NB, NB)])
        pltpu.sync_copy(accmn, omin.at[pl.ds(nb * NB, NB)])

    def blk_loop(i, _):
        nb = wid + NW * i

        @pl.when(nb < NBLK)
        def _():
            do_block(nb)
        return 0

    lax.fori_loop(0, WBLKS, blk_loop, 0)


def _sc_agg(xr, rel, srcp, tpm, blkb):
    zer = jnp.zeros((NB, B * D), jnp.float32)
    ninf = jnp.full((NB, B * D), NEG, jnp.float32)
    pinf = jnp.full((NB, B * D), -NEG, jnp.float32)
    mesh = plsc.VectorSubcoreMesh(core_axis_name="c", subcore_axis_name="s")
    f = pl.kernel(
        _sc_body,
        out_type=[
            jax.ShapeDtypeStruct((NPAD, B * D), jnp.float32),
            jax.ShapeDtypeStruct((NPAD, B * D), jnp.float32),
            jax.ShapeDtypeStruct((NPAD, B * D), jnp.float32),
            jax.ShapeDtypeStruct((NPAD, B * D), jnp.float32),
        ],
        mesh=mesh,
        scratch_types=[
            pltpu.VMEM((NB, B * D), jnp.float32),
            pltpu.VMEM((NB, B * D), jnp.float32),
            pltpu.VMEM((NB, B * D), jnp.float32),
            pltpu.VMEM((NB, B * D), jnp.float32),
            pltpu.VMEM((G, B * D), jnp.float32),
            pltpu.VMEM((G, B * D), jnp.float32),
            pltpu.VMEM((64, B * D), jnp.float32),
            pltpu.VMEM((G,), jnp.int32),
            pltpu.VMEM((16,), jnp.int32),
            pltpu.SemaphoreType.DMA,
        ],
    )
    return f(xr, rel, srcp, tpm, blkb, zer, ninf, pinf)


def _combine_body(osum_ref, osq_ref, omax_ref, omin_ref, scl_ref,
                  x_ref, bnd_ref, W_ref, bl_ref, out0_ref, out1_ref):
    bnd = bnd_ref[...]
    c = scl_ref[:, 0:1]
    s = osum_ref[...] + bnd
    q = osq_ref[...] + bnd * bnd
    mx = jnp.maximum(omax_ref[...], bnd)
    mn = jnp.minimum(omin_ref[...], bnd)
    mean = s / c
    std = jnp.sqrt(jnp.maximum(q / c - mean * mean, EPS))
    sc1 = scl_ref[:, 1:2]
    sc2 = scl_ref[:, 2:3]
    for b, oref in ((0, out0_ref), (1, out1_ref)):
        sl = slice(b * D, (b + 1) * D)
        parts = []
        for st in (mean, mx, mn, std):
            stb = st[:, sl]
            parts.extend([stb, stb * sc1, stb * sc2])
        z = jnp.concatenate(parts + [x_ref[:, sl]], axis=1)
        ob = jnp.dot(z, W_ref[...], preferred_element_type=jnp.float32)
        oref[...] = jnp.maximum(ob + bl_ref[...], 0.0)


def _combine(osum, osq, omax, omin, scl, xpad, bpad, W_perm, b_lin2):
    blk = pl.BlockSpec((NB, B * D), lambda i: (i, 0))
    return pl.pallas_call(
        _combine_body,
        grid=(NBLK,),
        in_specs=[
            blk, blk, blk, blk,
            pl.BlockSpec((NB, 8), lambda i: (i, 0)),
            blk, blk,
            pl.BlockSpec(((12 + 1) * D, OUT), lambda i: (0, 0)),
            pl.BlockSpec((1, OUT), lambda i: (0, 0)),
        ],
        out_specs=[
            pl.BlockSpec((NB, OUT), lambda i: (i, 0)),
            pl.BlockSpec((NB, OUT), lambda i: (i, 0)),
        ],
        out_shape=[
            jax.ShapeDtypeStruct((NPAD, OUT), jnp.float32),
            jax.ShapeDtypeStruct((NPAD, OUT), jnp.float32),
        ],
    )(osum, osq, omax, omin, scl, xpad, bpad, W_perm, b_lin2)


def kernel(x, boundary, query, edge_weight, edge_index, edge_type, edge_time,
           query_time, W_rel, b_rel, W_time, b_time, W_lin, b_lin):
    src = edge_index[0].astype(jnp.int32)
    dst = edge_index[1].astype(jnp.int32)
    et = edge_type.astype(jnp.int32)
    tm = edge_time.astype(jnp.int32)

    perm = jnp.argsort(dst)
    srcp = jnp.pad(src[perm], (0, EP - E))
    dstp = dst[perm]
    typep = jnp.pad(et[perm], (0, EP - E))
    tmp_ = jnp.pad(tm[perm], (0, EP - E))
    dstloc = jnp.pad(dstp % NB, (0, EP - E))
    blkb1 = jnp.searchsorted(dstp, jnp.arange(0, NPAD + NB, NB)).astype(jnp.int32)
    blkb = jnp.stack([blkb1[:NBLK], blkb1[1:NBLK + 1]], axis=1)
    blkb = jnp.pad(blkb, ((0, 0), (0, 14)))

    jp = jnp.arange(12 * D)
    k_, s_, d_ = jp // (3 * D), (jp // D) % 3, jp % D
    W_perm = jnp.concatenate([W_lin[d_ * 12 + k_ * 3 + s_], W_lin[12 * D:]], axis=0)

    tcol = tmp_.astype(jnp.float32).reshape(EP, 1)
    dcol = dstloc.astype(jnp.float32).reshape(EP, 1)
    ycol = typep.astype(jnp.float32).reshape(EP, 1)
    qt3 = jnp.pad(query_time.astype(jnp.float32), (0, 8 - B)).reshape(1, 1, 8)
    b_time2 = b_time.reshape(1, D)
    b_rel2 = b_rel.reshape(R, D)

    tpm, rel = _edge_prep(tcol, dcol, ycol, qt3, W_time, b_time2,
                          query, W_rel, b_rel2)

    xr = x.reshape(N, B * D)
    osum, osq, omax, omin = _sc_agg(xr, rel, srcp, tpm, blkb)

    node_b = jnp.searchsorted(dstp, jnp.arange(0, N + 1)).astype(jnp.int32)
    cnt = (node_b[1:] - node_b[:-1]).astype(jnp.float32) + 1.0
    logd = jnp.log(cnt)
    scale = logd / jnp.mean(logd)
    inv = 1.0 / jnp.maximum(scale, 1e-2)
    scl = jnp.zeros((NPAD, 8), jnp.float32)
    scl = scl.at[:, 0].set(1.0)
    scl = scl.at[:N, 0].set(cnt)
    scl = scl.at[:N, 1].set(scale)
    scl = scl.at[:N, 2].set(inv)

    xpad = jnp.pad(xr, ((0, NPAD - N), (0, 0)))
    bpad = jnp.pad(boundary.reshape(N, B * D), ((0, NPAD - N), (0, 0)))
    out0, out1 = _combine(osum, osq, omax, omin, scl, xpad, bpad,
                          W_perm, b_lin.reshape(1, OUT))
    return jnp.stack([out0[:N], out1[:N]], axis=1)
